# baseline (speedup 1.0000x reference)
import jax
import jax.numpy as jnp
from jax import lax
from jax.experimental import pallas as pl
from jax.experimental.pallas import tpu as pltpu
from jax.experimental.pallas import tpu_sc as plsc

N_ATOMS = 100000
ATOM_DIM = 128
HALF = ATOM_DIM // 2
N_FGS = 20000
TOTAL = 320000

NTILES = 16
CH = 96
NCHUNK = 216
GS = 24
PER_TILE = NCHUNK * CH
PAD_TOTAL = PER_TILE * NTILES
ACC_ROWS = 20008
CNT_ROWS = 20016
WBA = 1256
WBL = N_FGS - 15 * WBA
NP = 20480


def _sc_body(atomL, atomR, ci, cs, ei, es, z64,
             csumL, csumR, esumL, esumR, ccnt_part, ecnt_part,
             idx_v, seg_v, braw0, braw1, braw2, braw3,
             rows_v, cnt_v, acc,
             gsem0, gsem1, gsem2, gsem3):
    bbufs = (braw0, braw1, braw2, braw3)
    gsems = (gsem0, gsem1, gsem2, gsem3)
    c = lax.axis_index("c")
    t = lax.axis_index("s")
    base = t * WBA

    def per_tile_slice(fn):
        @pl.when(t < NTILES - 1)
        def _():
            fn(pl.ds(base, WBA), WBA)

        @pl.when(t == NTILES - 1)
        def _():
            fn(pl.ds(base, WBL), WBL)

    def zero_acc():
        per_tile_slice(lambda s, n: pltpu.sync_copy(z64.at[pl.ds(0, n)], acc.at[s]))

    def zero_cnt(_j, carry):
        cnt_v[pl.ds(_j * 16, 16)] = jnp.zeros((16,), jnp.float32)
        return carry

    zero_acc()
    plsc.subcore_barrier()

    def do_phase(idx_hbm, seg_hbm, outL, outR, cnt_out, cnt_core):
        @pl.when(c == cnt_core)
        def _():
            lax.fori_loop(0, CNT_ROWS // 16, zero_cnt, 0)

        ones16 = jnp.ones((16,), jnp.float32)

        def gather_issue(j, buf, s):
            @pl.when(c == 0)
            def _():
                pltpu.async_copy(atomL.at[idx_v.at[j]], buf, s)

            @pl.when(c == 1)
            def _():
                pltpu.async_copy(atomR.at[idx_v.at[j]], buf, s)

        def gather_wait(buf, s):
            pltpu.make_async_copy(atomL.at[idx_v.at[0]], buf, s).wait()

        def widen(u):
            bb = bbufs[u]
            fb = rows_v
            hi_mask = jnp.full((16,), -65536, jnp.int32)

            def row(r, carry):
                for b in range(2):
                    v = bb[r, pl.ds(b * 16, 16)]
                    lo = plsc.bitcast(lax.shift_left(v, 16), jnp.float32)
                    hi = plsc.bitcast(lax.bitwise_and(v, hi_mask), jnp.float32)
                    fb[r, pl.ds(b * 32, 16)] = lo
                    fb[r, pl.ds(b * 32 + 16, 16)] = hi
                return carry

            lax.fori_loop(0, CH, row, 0)

        def retire(jj, u, gjj):
            gather_wait(bbufs[u], gsems[u])
            widen(u)
            pltpu.sync_copy(rows_v, acc.at[seg_v.at[jj]], add=True)

            @pl.when(c == cnt_core)
            def _():
                for k in range(CH // 16):
                    seg16 = seg_v[jj, pl.ds(k * 16, 16)]
                    plsc.addupdate_scatter(cnt_v, [seg16], ones16)

        def group(g, carry):
            grows = pl.ds(t * NCHUNK + g * GS, GS)
            pltpu.sync_copy(idx_hbm.at[grows], idx_v)
            pltpu.sync_copy(seg_hbm.at[grows], seg_v)

            def quad(i, carry2):
                for u in range(4):
                    j = 4 * i + u
                    gather_issue(j, bbufs[u], gsems[u])

                    @pl.when(j >= 2)
                    def _():
                        retire(j - 2, (u + 2) % 4, g * GS + j - 2)

                return carry2

            lax.fori_loop(0, GS // 4, quad, 0)
            retire(GS - 2, 2, g * GS + GS - 2)
            retire(GS - 1, 3, g * GS + GS - 1)
            return carry

        lax.fori_loop(0, NCHUNK // GS, group, 0)

        @pl.when(c == cnt_core)
        def _():
            def wcnt(i, carry):
                s = pl.ds(i * 2000, 2000)
                pltpu.sync_copy(cnt_v.at[s], cnt_out.at[t, s])
                return carry

            lax.fori_loop(0, N_FGS // 2000, wcnt, 0)

        plsc.subcore_barrier()

        def writeback(s, n):
            @pl.when(c == 0)
            def _():
                pltpu.sync_copy(acc.at[s], outL.at[s])

            @pl.when(c == 1)
            def _():
                pltpu.sync_copy(acc.at[s], outR.at[s])

        per_tile_slice(writeback)

    do_phase(ci, cs, csumL, csumR, ccnt_part, 0)
    zero_acc()
    plsc.subcore_barrier()
    do_phase(ei, es, esumL, esumR, ecnt_part, 1)


def _tc_body(csL, csR, esL, esR, cc, ec, w1a, w1b, b1, w2, b2, out):
    ones_col = jnp.ones((NTILES, 1), jnp.float32)
    dims = (((0,), (0,)), ((), ()))
    cc0 = lax.dot_general(cc[:], ones_col, dims,
                          preferred_element_type=jnp.float32)
    ec0 = lax.dot_general(ec[:], ones_col, dims,
                          preferred_element_type=jnp.float32)
    cm = jnp.concatenate([csL[:], csR[:]], axis=1) / jnp.maximum(cc0, 1.0)
    em = jnp.concatenate([esL[:], esR[:]], axis=1) / jnp.maximum(ec0, 1.0)
    h = jnp.dot(cm, w1a[:], preferred_element_type=jnp.float32)
    h = h + jnp.dot(em, w1b[:], preferred_element_type=jnp.float32)
    h = jnp.maximum(h + b1[:], 0.0)
    out[:] = jnp.dot(h, w2[:], preferred_element_type=jnp.float32) + b2[:]


def kernel(atom_feats, core_idx, core_seg, env_idx, env_seg, W1, b1, W2, b2):
    f32 = jnp.float32
    sigma = []
    for blk in range(HALF // 32):
        for k in range(16):
            sigma.extend([blk * 32 + k, blk * 32 + 16 + k])
    sigma = jnp.asarray(sigma, jnp.int32)
    atom_bf = atom_feats.astype(jnp.bfloat16)

    def pack_half(tab):
        perm = jnp.take(tab, sigma, axis=1)
        return lax.bitcast_convert_type(
            perm.reshape(N_ATOMS, HALF // 2, 2), jnp.int32)

    atomL = pack_half(atom_bf[:, :HALF])
    atomR = pack_half(atom_bf[:, HALF:])
    pad = PAD_TOTAL - TOTAL
    ci = jnp.concatenate([core_idx, jnp.zeros((pad,), jnp.int32)]).reshape(-1, CH)
    cs = jnp.concatenate([core_seg, jnp.full((pad,), N_FGS, jnp.int32)]).reshape(-1, CH)
    ei = jnp.concatenate([env_idx, jnp.zeros((pad,), jnp.int32)]).reshape(-1, CH)
    es = jnp.concatenate([env_seg, jnp.full((pad,), N_FGS, jnp.int32)]).reshape(-1, CH)
    z64 = jnp.zeros((WBA, HALF), f32)

    mesh = plsc.VectorSubcoreMesh(core_axis_name="c", subcore_axis_name="s")
    sc = pl.kernel(
        _sc_body,
        out_type=[
            jax.ShapeDtypeStruct((NP, HALF), f32),
            jax.ShapeDtypeStruct((NP, HALF), f32),
            jax.ShapeDtypeStruct((NP, HALF), f32),
            jax.ShapeDtypeStruct((NP, HALF), f32),
            jax.ShapeDtypeStruct((NTILES, NP), f32),
            jax.ShapeDtypeStruct((NTILES, NP), f32),
        ],
        mesh=mesh,
        scratch_types=[
            pltpu.VMEM((GS, CH), jnp.int32),
            pltpu.VMEM((GS, CH), jnp.int32),
            pltpu.VMEM((CH, HALF // 2), jnp.int32),
            pltpu.VMEM((CH, HALF // 2), jnp.int32),
            pltpu.VMEM((CH, HALF // 2), jnp.int32),
            pltpu.VMEM((CH, HALF // 2), jnp.int32),
            pltpu.VMEM((CH, HALF), f32),
            pltpu.VMEM((CNT_ROWS,), f32),
            pltpu.VMEM_SHARED((ACC_ROWS, HALF), f32),
            pltpu.SemaphoreType.DMA,
            pltpu.SemaphoreType.DMA,
            pltpu.SemaphoreType.DMA,
            pltpu.SemaphoreType.DMA,
        ],
        compiler_params=pltpu.CompilerParams(
            use_tc_tiling_on_sc=False, needs_layout_passes=False),
    )
    csumL, csumR, esumL, esumR, ccnt_part, ecnt_part = sc(
        atomL, atomR, ci, cs, ei, es, z64)

    BT = 1024
    grid = (NP // BT,)
    row_spec = pl.BlockSpec((BT, HALF), lambda i: (i, 0))
    cnt_spec = pl.BlockSpec((NTILES, BT), lambda i: (0, i))
    w_spec = pl.BlockSpec((ATOM_DIM, ATOM_DIM), lambda i: (0, 0))
    b_spec = pl.BlockSpec((1, ATOM_DIM), lambda i: (0, 0))
    out = pl.pallas_call(
        _tc_body,
        grid=grid,
        in_specs=[row_spec, row_spec, row_spec, row_spec, cnt_spec, cnt_spec,
                  w_spec, w_spec, b_spec, w_spec, b_spec],
        out_specs=pl.BlockSpec((BT, ATOM_DIM), lambda i: (i, 0)),
        out_shape=jax.ShapeDtypeStruct((NP, ATOM_DIM), f32),
    )(csumL, csumR, esumL, esumR, ccnt_part, ecnt_part,
      W1[:ATOM_DIM], W1[ATOM_DIM:], b1.reshape(1, -1), W2, b2.reshape(1, -1))
    return out[:N_FGS]


# speedup vs baseline: 1.3823x; 1.3823x over previous
import jax
import jax.numpy as jnp
from jax import lax
from jax.experimental import pallas as pl
from jax.experimental.pallas import tpu as pltpu
from jax.experimental.pallas import tpu_sc as plsc

N_ATOMS = 100000
ATOM_DIM = 128
HALF = ATOM_DIM // 2
N_FGS = 20000
TOTAL = 320000

NTILES = 16
CH = 96
NCHUNK = 216
GS = 24
PER_TILE = NCHUNK * CH
PAD_TOTAL = PER_TILE * NTILES
ACC_ROWS = 20008
CNT_ROWS = 20016
WBA = 1256
WBL = N_FGS - 15 * WBA
NP = 20480


def _sc_body(atomL, atomR, ci, cs, ei, es, z64,
             csumL, csumR, esumL, esumR, ccnt_part, ecnt_part,
             idx_v, seg_v, rows0, rows1, rows2, rows3, cnt_v, acc,
             gsem0, gsem1, gsem2, gsem3, ssem0, ssem1, ssem2, ssem3):
    rbufs = (rows0, rows1, rows2, rows3)
    gsems = (gsem0, gsem1, gsem2, gsem3)
    ssems = (ssem0, ssem1, ssem2, ssem3)
    c = lax.axis_index("c")
    t = lax.axis_index("s")
    base = t * WBA

    def per_tile_slice(fn):
        @pl.when(t < NTILES - 1)
        def _():
            fn(pl.ds(base, WBA), WBA)

        @pl.when(t == NTILES - 1)
        def _():
            fn(pl.ds(base, WBL), WBL)

    def zero_acc():
        per_tile_slice(lambda s, n: pltpu.sync_copy(z64.at[pl.ds(0, n)], acc.at[s]))

    def zero_cnt(_j, carry):
        cnt_v[pl.ds(_j * 16, 16)] = jnp.zeros((16,), jnp.float32)
        return carry

    zero_acc()
    plsc.subcore_barrier()

    def do_phase(idx_hbm, seg_hbm, outL, outR, cnt_out, cnt_core):
        @pl.when(c == cnt_core)
        def _():
            lax.fori_loop(0, CNT_ROWS // 16, zero_cnt, 0)

        ones16 = jnp.ones((16,), jnp.float32)

        def gather_issue(j, buf, s):
            @pl.when(c == 0)
            def _():
                pltpu.async_copy(atomL.at[idx_v.at[j]], buf, s)

            @pl.when(c == 1)
            def _():
                pltpu.async_copy(atomR.at[idx_v.at[j]], buf, s)

        def gather_wait(buf, s):
            pltpu.make_async_copy(atomL.at[idx_v.at[0]], buf, s).wait()

        def scatter_wait(u):
            pltpu.make_async_copy(rbufs[u], acc.at[pl.ds(0, CH)], ssems[u]).wait()

        def retire(jj, u):
            gather_wait(rbufs[u], gsems[u])
            pltpu.async_copy(rbufs[u], acc.at[seg_v.at[jj]], ssems[u], add=True)

            @pl.when(c == cnt_core)
            def _():
                for k in range(CH // 16):
                    seg16 = seg_v[jj, pl.ds(k * 16, 16)]
                    plsc.addupdate_scatter(cnt_v, [seg16], ones16)

        def group(g, carry):
            grows = pl.ds(t * NCHUNK + g * GS, GS)
            pltpu.sync_copy(idx_hbm.at[grows], idx_v)
            pltpu.sync_copy(seg_hbm.at[grows], seg_v)

            def quad(i, carry2):
                for u in range(4):
                    j = 4 * i + u

                    @pl.when(g * GS + j >= 4)
                    def _():
                        scatter_wait(u)

                    gather_issue(j, rbufs[u], gsems[u])

                    @pl.when(j >= 3)
                    def _():
                        retire(j - 3, (u + 1) % 4)

                return carry2

            lax.fori_loop(0, GS // 4, quad, 0)
            retire(GS - 3, 1)
            retire(GS - 2, 2)
            retire(GS - 1, 3)
            return carry

        lax.fori_loop(0, NCHUNK // GS, group, 0)
        for u in range(4):
            scatter_wait(u)

        @pl.when(c == cnt_core)
        def _():
            def wcnt(i, carry):
                s = pl.ds(i * 2000, 2000)
                pltpu.sync_copy(cnt_v.at[s], cnt_out.at[t, s])
                return carry

            lax.fori_loop(0, N_FGS // 2000, wcnt, 0)

        plsc.subcore_barrier()

        def writeback(s, n):
            @pl.when(c == 0)
            def _():
                pltpu.sync_copy(acc.at[s], outL.at[s])

            @pl.when(c == 1)
            def _():
                pltpu.sync_copy(acc.at[s], outR.at[s])

        per_tile_slice(writeback)

    do_phase(ci, cs, csumL, csumR, ccnt_part, 0)
    zero_acc()
    plsc.subcore_barrier()
    do_phase(ei, es, esumL, esumR, ecnt_part, 1)


def _tc_body(csL, csR, esL, esR, cc, ec, w1a, w1b, b1, w2, b2, out):
    ones_col = jnp.ones((NTILES, 1), jnp.float32)
    dims = (((0,), (0,)), ((), ()))
    cc0 = lax.dot_general(cc[:], ones_col, dims,
                          preferred_element_type=jnp.float32)
    ec0 = lax.dot_general(ec[:], ones_col, dims,
                          preferred_element_type=jnp.float32)
    cm = jnp.concatenate([csL[:], csR[:]], axis=1) / jnp.maximum(cc0, 1.0)
    em = jnp.concatenate([esL[:], esR[:]], axis=1) / jnp.maximum(ec0, 1.0)
    h = jnp.dot(cm, w1a[:], preferred_element_type=jnp.float32)
    h = h + jnp.dot(em, w1b[:], preferred_element_type=jnp.float32)
    h = jnp.maximum(h + b1[:], 0.0)
    out[:] = jnp.dot(h, w2[:], preferred_element_type=jnp.float32) + b2[:]


def kernel(atom_feats, core_idx, core_seg, env_idx, env_seg, W1, b1, W2, b2):
    f32 = jnp.float32
    atomL = atom_feats[:, :HALF]
    atomR = atom_feats[:, HALF:]
    pad = PAD_TOTAL - TOTAL
    ci = jnp.concatenate([core_idx, jnp.zeros((pad,), jnp.int32)]).reshape(-1, CH)
    cs = jnp.concatenate([core_seg, jnp.full((pad,), N_FGS, jnp.int32)]).reshape(-1, CH)
    ei = jnp.concatenate([env_idx, jnp.zeros((pad,), jnp.int32)]).reshape(-1, CH)
    es = jnp.concatenate([env_seg, jnp.full((pad,), N_FGS, jnp.int32)]).reshape(-1, CH)
    z64 = jnp.zeros((WBA, HALF), f32)

    mesh = plsc.VectorSubcoreMesh(core_axis_name="c", subcore_axis_name="s")
    sc = pl.kernel(
        _sc_body,
        out_type=[
            jax.ShapeDtypeStruct((NP, HALF), f32),
            jax.ShapeDtypeStruct((NP, HALF), f32),
            jax.ShapeDtypeStruct((NP, HALF), f32),
            jax.ShapeDtypeStruct((NP, HALF), f32),
            jax.ShapeDtypeStruct((NTILES, NP), f32),
            jax.ShapeDtypeStruct((NTILES, NP), f32),
        ],
        mesh=mesh,
        scratch_types=[
            pltpu.VMEM((GS, CH), jnp.int32),
            pltpu.VMEM((GS, CH), jnp.int32),
            pltpu.VMEM((CH, HALF), f32),
            pltpu.VMEM((CH, HALF), f32),
            pltpu.VMEM((CH, HALF), f32),
            pltpu.VMEM((CH, HALF), f32),
            pltpu.VMEM((CNT_ROWS,), f32),
            pltpu.VMEM_SHARED((ACC_ROWS, HALF), f32),
            pltpu.SemaphoreType.DMA,
            pltpu.SemaphoreType.DMA,
            pltpu.SemaphoreType.DMA,
            pltpu.SemaphoreType.DMA,
            pltpu.SemaphoreType.DMA,
            pltpu.SemaphoreType.DMA,
            pltpu.SemaphoreType.DMA,
            pltpu.SemaphoreType.DMA,
        ],
        compiler_params=pltpu.CompilerParams(
            use_tc_tiling_on_sc=False, needs_layout_passes=False),
    )
    csumL, csumR, esumL, esumR, ccnt_part, ecnt_part = sc(
        atomL, atomR, ci, cs, ei, es, z64)

    BT = 1024
    grid = (NP // BT,)
    row_spec = pl.BlockSpec((BT, HALF), lambda i: (i, 0))
    cnt_spec = pl.BlockSpec((NTILES, BT), lambda i: (0, i))
    w_spec = pl.BlockSpec((ATOM_DIM, ATOM_DIM), lambda i: (0, 0))
    b_spec = pl.BlockSpec((1, ATOM_DIM), lambda i: (0, 0))
    out = pl.pallas_call(
        _tc_body,
        grid=grid,
        in_specs=[row_spec, row_spec, row_spec, row_spec, cnt_spec, cnt_spec,
                  w_spec, w_spec, b_spec, w_spec, b_spec],
        out_specs=pl.BlockSpec((BT, ATOM_DIM), lambda i: (i, 0)),
        out_shape=jax.ShapeDtypeStruct((NP, ATOM_DIM), f32),
    )(csumL, csumR, esumL, esumR, ccnt_part, ecnt_part,
      W1[:ATOM_DIM], W1[ATOM_DIM:], b1.reshape(1, -1), W2, b2.reshape(1, -1))
    return out[:N_FGS]
